# Initial kernel scaffold; baseline (speedup 1.0000x reference)
import jax
import jax.numpy as jnp
from jax.experimental import pallas as pl


def kernel(feat, loc, edge_index, mid, W2, Wd, Ww1, Ww2, va, dist_emb, boundaries):
    raise NotImplementedError("write your pallas kernel here")


# speedup vs baseline: 2.9866x; 2.9866x over previous
import functools

import jax
import jax.numpy as jnp
from jax import lax
from jax.experimental import pallas as pl
from jax.experimental.pallas import tpu as pltpu
from jax.experimental.pallas import tpu_sc as plsc

N = 10000
NPAD = 10240
E = 320000
D = 128
NB = 14
QUART = 2500
ACC_ROWS = 2688
STRIPE = ACC_ROWS // 16
EPT = E // 16
K = 80
NCH = EPT // K
EPW = E // 32
NCHD = EPW // K

_mesh = plsc.VectorSubcoreMesh(core_axis_name="c", subcore_axis_name="s")
_sc_params = pltpu.CompilerParams(needs_layout_passes=False)


@functools.partial(
    pl.kernel,
    out_type=jax.ShapeDtypeStruct((32 * NPAD,), jnp.float32),
    mesh=_mesh,
    scratch_types=[
        pltpu.VMEM((K,), jnp.int32),
        pltpu.VMEM((NPAD,), jnp.float32),
    ],
    compiler_params=_sc_params,
)
def _deg_kernel(src_h, degp_h, idx_v, acc_v):
    cid = lax.axis_index("c")
    sid = lax.axis_index("s")
    wid = cid * 16 + sid
    zf = jnp.zeros((16,), jnp.float32)
    onef = jnp.ones((16,), jnp.float32)
    lane = lax.iota(jnp.int32, 16)

    @pl.loop(0, NPAD // 16)
    def _zero(i):
        acc_v[pl.ds(i * 16, 16)] = zf

    base0 = wid * EPW

    @pl.loop(0, NCHD)
    def _chunk(j):
        pltpu.sync_copy(src_h.at[pl.ds(base0 + j * K, K)], idx_v)
        for g in range(K // 16):
            iv = idx_v[pl.ds(g * 16, 16)]
            for l in range(16):
                plsc.addupdate_scatter(acc_v, [iv], onef, mask=lane == l)

    pltpu.sync_copy(acc_v, degp_h.at[pl.ds(wid * NPAD, NPAD)])


def _dense_body(feat_ref, w2t_ref, u3_ref, degt_ref, h2_ref, scal_ref):
    h2 = jnp.dot(feat_ref[...], w2t_ref[...], preferred_element_type=jnp.float32)
    h2_ref[...] = h2
    s = jnp.dot(h2, u3_ref[...], preferred_element_type=jnp.float32)
    deg = jnp.sum(degt_ref[...], axis=1, keepdims=True)
    d0 = lax.rsqrt(jnp.maximum(deg, 1.0))
    scal_ref[...] = jnp.concatenate([s, d0], axis=1)


_dense_call = pl.pallas_call(
    _dense_body,
    out_shape=(
        jax.ShapeDtypeStruct((N, D), jnp.float32),
        jax.ShapeDtypeStruct((N, 4), jnp.float32),
    ),
)


_edge_scratch = [
        pltpu.VMEM((4 * N,), jnp.float32),
        pltpu.VMEM((2 * N,), jnp.float32),
        pltpu.VMEM((16,), jnp.float32),
        pltpu.VMEM((NB, 16), jnp.float32),
        pltpu.VMEM((K,), jnp.int32),
        pltpu.VMEM((K,), jnp.int32),
        pltpu.VMEM((K,), jnp.int32),
        pltpu.VMEM((K,), jnp.float32),
        pltpu.VMEM((K,), jnp.float32),
        pltpu.VMEM((K, D), jnp.float32),
        pltpu.VMEM((K, D), jnp.float32),
        pltpu.VMEM((K, D), jnp.float32),
        pltpu.VMEM_SHARED((ACC_ROWS, D), jnp.float32),
        pltpu.SemaphoreType.DMA,
]


def _edge_body(lo, src_h, dst_h, mid_h, locf_h, h2_h, scalf_h, t_h, b2_h,
               part_h,
               scalf, locf, t_v, b2_v, srcb, dstb, midb, cabuf, cbbuf,
               asrc, amid, hebuf, y_acc, sem):
    cid = lax.axis_index("c")
    sid = lax.axis_index("s")

    pltpu.sync_copy(scalf_h, scalf)
    pltpu.sync_copy(locf_h, locf)
    pltpu.sync_copy(t_h, t_v)
    pltpu.sync_copy(b2_h, b2_v)
    zf = jnp.zeros((16,), jnp.float32)

    @pl.loop(0, K * D // 16)
    def _zb(i):
        hebuf[i // (D // 16), pl.ds((i % (D // 16)) * 16, 16)] = zf

    for r in range(STRIPE // K):
        pltpu.sync_copy(hebuf, y_acc.at[pl.ds(sid * STRIPE + r * K, K)])
    pltpu.sync_copy(hebuf.at[pl.ds(0, STRIPE % K)],
                    y_acc.at[pl.ds(sid * STRIPE + (STRIPE // K) * K, STRIPE % K)])
    plsc.subcore_barrier()

    base0 = sid * EPT
    lobase = lo + cid * QUART
    c0 = jnp.zeros((16,), jnp.int32)
    c1 = jnp.full((16,), 1, jnp.int32)
    c2 = jnp.full((16,), 2, jnp.int32)
    c3 = jnp.full((16,), 3, jnp.int32)
    chalf = jnp.full((16,), QUART, jnp.int32)
    cmask = jnp.full((16,), 127, jnp.int32)

    @pl.loop(0, NCH)
    def _chunk(j):
        base = base0 + j * K
        pltpu.sync_copy(src_h.at[pl.ds(base, K)], srcb)
        pltpu.sync_copy(dst_h.at[pl.ds(base, K)], dstb)
        pltpu.sync_copy(mid_h.at[pl.ds(base, K)], midb)
        cp1 = pltpu.async_copy(h2_h.at[srcb], asrc, sem)
        cp2 = pltpu.async_copy(h2_h.at[midb], amid, sem)
        cp1.wait()
        cp2.wait()

        for g in range(K // 16):
            sv = srcb[pl.ds(g * 16, 16)]
            dv = dstb[pl.ds(g * 16, 16)]
            mv = midb[pl.ds(g * 16, 16)]
            sv2 = sv + sv
            dv2 = dv + dv
            lxs = plsc.load_gather(locf, [sv2])
            lys = plsc.load_gather(locf, [sv2 + c1])
            lxd = plsc.load_gather(locf, [dv2])
            lyd = plsc.load_gather(locf, [dv2 + c1])
            dx = lxd - lxs
            dy = lyd - lys
            d2 = dx * dx + dy * dy
            cnt = c0
            for q in range(NB):
                cnt = cnt + (b2_v[q] < d2).astype(jnp.int32)
            tv = plsc.load_gather(t_v, [cnt])
            sv4 = sv2 + sv2
            dv4 = dv2 + dv2
            mv4 = (mv + mv) + (mv + mv)
            s1d = plsc.load_gather(scalf, [dv4])
            s3m = plsc.load_gather(scalf, [mv4 + c1])
            s4s = plsc.load_gather(scalf, [sv4 + c2])
            d0s = plsc.load_gather(scalf, [sv4 + c3])
            score = s1d + s4s + s3m + tv
            beta = 1.0 / (1.0 + jnp.exp(-score))
            cabuf[pl.ds(g * 16, 16)] = d0s * beta
            cbbuf[pl.ds(g * 16, 16)] = d0s
            rel = dv - lobase
            own = (rel >= c0) & (rel < chalf)
            dstb[pl.ds(g * 16, 16)] = jnp.where(own, rel, chalf + (dv & cmask))

        @pl.loop(0, K)
        def _edge(k):
            kk = jnp.full((16,), k, jnp.int32)
            cav = plsc.load_gather(cabuf, [kk])
            cbv = plsc.load_gather(cbbuf, [kk])
            for s in range(D // 16):
                a = asrc[k, pl.ds(s * 16, 16)]
                b = amid[k, pl.ds(s * 16, 16)]
                hebuf[k, pl.ds(s * 16, 16)] = cav * a + cbv * b

        pltpu.sync_copy(hebuf, y_acc.at[dstb], add=True)

    plsc.subcore_barrier()
    pltpu.sync_copy(y_acc.at[pl.ds(sid * STRIPE, STRIPE)],
                    part_h.at[cid, pl.ds(sid * STRIPE, STRIPE)])


_edge_kernel_a = functools.partial(
    pl.kernel,
    out_type=jax.ShapeDtypeStruct((2, ACC_ROWS, D), jnp.float32),
    mesh=_mesh,
    scratch_types=_edge_scratch,
    compiler_params=_sc_params,
)(functools.partial(_edge_body, 0))

_edge_kernel_b = functools.partial(
    pl.kernel,
    out_type=jax.ShapeDtypeStruct((2, ACC_ROWS, D), jnp.float32),
    mesh=_mesh,
    scratch_types=_edge_scratch,
    compiler_params=_sc_params,
)(functools.partial(_edge_body, 2 * QUART))


def _fin_body(parta_ref, partb_ref, scal_ref, x_ref):
    y = jnp.concatenate(
        [parta_ref[0, :QUART, :], parta_ref[1, :QUART, :],
         partb_ref[0, :QUART, :], partb_ref[1, :QUART, :]], axis=0)
    x_ref[...] = scal_ref[:, 3:4] * y


_fin_call = pl.pallas_call(
    _fin_body,
    out_shape=jax.ShapeDtypeStruct((N, D), jnp.float32),
)


def kernel(feat, loc, edge_index, mid, W2, Wd, Ww1, Ww2, va, dist_emb, boundaries):
    src = edge_index[0]
    dst = edge_index[1]
    u = va[0] @ Ww1
    v = va[0] @ Ww2
    u1 = u[:D]
    u2 = u[D:]
    u3 = v[:D]
    u4 = v[D:]
    t = (dist_emb @ Wd.T) @ u2
    t16 = jnp.pad(t, (0, 1))
    b2 = jnp.broadcast_to((boundaries * boundaries)[:, None], (NB, 16))
    U3 = jnp.stack([u1, u3, u4], axis=1)
    degp = _deg_kernel(src).reshape(32, NPAD)[:, :N].T
    h2, scal = _dense_call(feat, W2.T, U3, degp)
    locf = loc.reshape(-1)
    scalf = scal.reshape(-1)
    parta = _edge_kernel_a(src, dst, mid, locf, h2, scalf, t16, b2)
    partb = _edge_kernel_b(src, dst, mid, locf, h2, scalf, t16, b2)
    return _fin_call(parta, partb, scal)


# speedup vs baseline: 4.1405x; 1.3864x over previous
import functools

import jax
import jax.numpy as jnp
from jax import lax
from jax.experimental import pallas as pl
from jax.experimental.pallas import tpu as pltpu
from jax.experimental.pallas import tpu_sc as plsc

N = 10000
NPAD = 10240
E = 320000
D = 128
NB = 14
QUART = 2500
ACC_ROWS = 2688
STRIPE = ACC_ROWS // 16
EPT = E // 16
K = 80
NCH = EPT // K
EPW = E // 32
NCHD = EPW // K

_mesh = plsc.VectorSubcoreMesh(core_axis_name="c", subcore_axis_name="s")
_sc_params = pltpu.CompilerParams(needs_layout_passes=False)


@functools.partial(
    pl.kernel,
    out_type=jax.ShapeDtypeStruct((32 * NPAD,), jnp.float32),
    mesh=_mesh,
    scratch_types=[
        pltpu.VMEM((K,), jnp.int32),
        pltpu.VMEM((NPAD,), jnp.float32),
    ],
    compiler_params=_sc_params,
)
def _deg_kernel(src_h, degp_h, idx_v, acc_v):
    cid = lax.axis_index("c")
    sid = lax.axis_index("s")
    wid = cid * 16 + sid
    zf = jnp.zeros((16,), jnp.float32)
    onef = jnp.ones((16,), jnp.float32)
    lane = lax.iota(jnp.int32, 16)

    @pl.loop(0, NPAD // 16)
    def _zero(i):
        acc_v[pl.ds(i * 16, 16)] = zf

    base0 = wid * EPW

    @pl.loop(0, NCHD)
    def _chunk(j):
        pltpu.sync_copy(src_h.at[pl.ds(base0 + j * K, K)], idx_v)
        for g in range(K // 16):
            iv = idx_v[pl.ds(g * 16, 16)]
            for l in range(16):
                plsc.addupdate_scatter(acc_v, [iv], onef, mask=lane == l)

    pltpu.sync_copy(acc_v, degp_h.at[pl.ds(wid * NPAD, NPAD)])


def _dense_body(feat_ref, w2t_ref, u3_ref, degt_ref, h2_ref, scal_ref):
    h2 = jnp.dot(feat_ref[...], w2t_ref[...], preferred_element_type=jnp.float32)
    h2_ref[...] = h2
    s = jnp.dot(h2, u3_ref[...], preferred_element_type=jnp.float32)
    deg = jnp.sum(degt_ref[...], axis=1, keepdims=True)
    d0 = lax.rsqrt(jnp.maximum(deg, 1.0))
    scal_ref[...] = jnp.concatenate([s, d0], axis=1)


_dense_call = pl.pallas_call(
    _dense_body,
    out_shape=(
        jax.ShapeDtypeStruct((N, D), jnp.float32),
        jax.ShapeDtypeStruct((N, 4), jnp.float32),
    ),
)


_edge_scratch = [
        pltpu.VMEM((4 * N,), jnp.float32),
        pltpu.VMEM((2 * N,), jnp.float32),
        pltpu.VMEM((16,), jnp.float32),
        pltpu.VMEM((NB, 16), jnp.float32),
        [pltpu.VMEM((K,), jnp.int32)] * 2,
        [pltpu.VMEM((K,), jnp.int32)] * 2,
        [pltpu.VMEM((K,), jnp.int32)] * 2,
        [pltpu.VMEM((K,), jnp.int32)] * 2,
        pltpu.VMEM((K,), jnp.float32),
        pltpu.VMEM((K,), jnp.float32),
        [pltpu.VMEM((K, D), jnp.float32)] * 2,
        [pltpu.VMEM((K, D), jnp.float32)] * 2,
        pltpu.VMEM_SHARED((ACC_ROWS, D), jnp.float32),
        [pltpu.SemaphoreType.DMA] * 2,
        [pltpu.SemaphoreType.DMA] * 2,
        [pltpu.SemaphoreType.DMA] * 2,
]


def _edge_body(lo, src_h, dst_h, mid_h, locf_h, h2_h, scalf_h, t_h, b2_h,
               part_h,
               scalf, locf, t_v, b2_v, srcb, dstb, midb, sdst, cabuf, cbbuf,
               asrc, amid, y_acc, semi, semg, sems):
    cid = lax.axis_index("c")
    sid = lax.axis_index("s")

    pltpu.sync_copy(scalf_h, scalf)
    pltpu.sync_copy(locf_h, locf)
    pltpu.sync_copy(t_h, t_v)
    pltpu.sync_copy(b2_h, b2_v)
    zf = jnp.zeros((16,), jnp.float32)

    @pl.loop(0, K * D // 16)
    def _zb(i):
        asrc[0][i // (D // 16), pl.ds((i % (D // 16)) * 16, 16)] = zf

    for r in range(STRIPE // K):
        pltpu.sync_copy(asrc[0], y_acc.at[pl.ds(sid * STRIPE + r * K, K)])
    pltpu.sync_copy(asrc[0].at[pl.ds(0, STRIPE % K)],
                    y_acc.at[pl.ds(sid * STRIPE + (STRIPE // K) * K, STRIPE % K)])
    plsc.subcore_barrier()

    base0 = sid * EPT
    lobase = lo + cid * QUART
    c0 = jnp.zeros((16,), jnp.int32)
    c1 = jnp.full((16,), 1, jnp.int32)
    c2 = jnp.full((16,), 2, jnp.int32)
    c3 = jnp.full((16,), 3, jnp.int32)
    chalf = jnp.full((16,), QUART, jnp.int32)
    cmask = jnp.full((16,), 127, jnp.int32)

    def _issue_idx(j, b):
        base = base0 + j * K
        pltpu.async_copy(src_h.at[pl.ds(base, K)], srcb[b], semi[b])
        pltpu.async_copy(dst_h.at[pl.ds(base, K)], dstb[b], semi[b])
        pltpu.async_copy(mid_h.at[pl.ds(base, K)], midb[b], semi[b])

    def _wait_idx(b):
        pltpu.make_async_copy(src_h.at[pl.ds(0, K)], srcb[b], semi[b]).wait()
        pltpu.make_async_copy(src_h.at[pl.ds(0, K)], dstb[b], semi[b]).wait()
        pltpu.make_async_copy(src_h.at[pl.ds(0, K)], midb[b], semi[b]).wait()

    def _issue_gather(b):
        pltpu.async_copy(h2_h.at[srcb[b]], asrc[b], semg[b])
        pltpu.async_copy(h2_h.at[midb[b]], amid[b], semg[b])

    def _wait_gather(b):
        pltpu.make_async_copy(h2_h.at[pl.ds(0, K)], asrc[b], semg[b]).wait()
        pltpu.make_async_copy(h2_h.at[pl.ds(0, K)], amid[b], semg[b]).wait()

    def _wait_scatter(b):
        pltpu.make_async_copy(h2_h.at[pl.ds(0, K)], asrc[b], sems[b]).wait()

    pltpu.sync_copy(src_h.at[pl.ds(base0, K)], srcb[0])
    pltpu.sync_copy(dst_h.at[pl.ds(base0, K)], dstb[0])
    pltpu.sync_copy(mid_h.at[pl.ds(base0, K)], midb[0])
    _issue_gather(0)

    @pl.loop(0, NCH // 2)
    def _pair(jj):
        for b in range(2):
            j = jj * 2 + b
            nb = 1 - b

            for g in range(K // 16):
                sv = srcb[b][pl.ds(g * 16, 16)]
                dv = dstb[b][pl.ds(g * 16, 16)]
                mv = midb[b][pl.ds(g * 16, 16)]
                sv2 = sv + sv
                dv2 = dv + dv
                lxs = plsc.load_gather(locf, [sv2])
                lys = plsc.load_gather(locf, [sv2 + c1])
                lxd = plsc.load_gather(locf, [dv2])
                lyd = plsc.load_gather(locf, [dv2 + c1])
                dx = lxd - lxs
                dy = lyd - lys
                d2 = dx * dx + dy * dy
                cnt = c0
                for q in range(NB):
                    cnt = cnt + (b2_v[q] < d2).astype(jnp.int32)
                tv = plsc.load_gather(t_v, [cnt])
                sv4 = sv2 + sv2
                dv4 = dv2 + dv2
                mv4 = (mv + mv) + (mv + mv)
                s1d = plsc.load_gather(scalf, [dv4])
                s3m = plsc.load_gather(scalf, [mv4 + c1])
                s4s = plsc.load_gather(scalf, [sv4 + c2])
                d0s = plsc.load_gather(scalf, [sv4 + c3])
                score = s1d + s4s + s3m + tv
                beta = 1.0 / (1.0 + jnp.exp(-score))
                cabuf[pl.ds(g * 16, 16)] = d0s * beta
                cbbuf[pl.ds(g * 16, 16)] = d0s
                rel = dv - lobase
                own = (rel >= c0) & (rel < chalf)
                sdst[b][pl.ds(g * 16, 16)] = jnp.where(
                    own, rel, chalf + (dv & cmask))

            @pl.when(j + 1 < NCH)
            def _pref_idx():
                _issue_idx(j + 1, nb)

            _wait_gather(b)

            @pl.loop(0, K)
            def _edge(k):
                kk = jnp.full((16,), k, jnp.int32)
                cav = plsc.load_gather(cabuf, [kk])
                cbv = plsc.load_gather(cbbuf, [kk])
                for s in range(D // 16):
                    a = asrc[b][k, pl.ds(s * 16, 16)]
                    bb = amid[b][k, pl.ds(s * 16, 16)]
                    asrc[b][k, pl.ds(s * 16, 16)] = cav * a + cbv * bb

            @pl.when(j + 1 < NCH)
            def _pref_gather():
                _wait_idx(nb)

                @pl.when(j >= 1)
                def _wsp():
                    _wait_scatter(nb)

                _issue_gather(nb)

            pltpu.async_copy(asrc[b], y_acc.at[sdst[b]], sems[b], add=True)

    _wait_scatter(0)
    _wait_scatter(1)
    plsc.subcore_barrier()
    pltpu.sync_copy(y_acc.at[pl.ds(sid * STRIPE, STRIPE)],
                    part_h.at[cid, pl.ds(sid * STRIPE, STRIPE)])


_edge_kernel_a = functools.partial(
    pl.kernel,
    out_type=jax.ShapeDtypeStruct((2, ACC_ROWS, D), jnp.float32),
    mesh=_mesh,
    scratch_types=_edge_scratch,
    compiler_params=_sc_params,
)(functools.partial(_edge_body, 0))

_edge_kernel_b = functools.partial(
    pl.kernel,
    out_type=jax.ShapeDtypeStruct((2, ACC_ROWS, D), jnp.float32),
    mesh=_mesh,
    scratch_types=_edge_scratch,
    compiler_params=_sc_params,
)(functools.partial(_edge_body, 2 * QUART))


def _fin_body(parta_ref, partb_ref, scal_ref, x_ref):
    y = jnp.concatenate(
        [parta_ref[0, :QUART, :], parta_ref[1, :QUART, :],
         partb_ref[0, :QUART, :], partb_ref[1, :QUART, :]], axis=0)
    x_ref[...] = scal_ref[:, 3:4] * y


_fin_call = pl.pallas_call(
    _fin_body,
    out_shape=jax.ShapeDtypeStruct((N, D), jnp.float32),
)


def kernel(feat, loc, edge_index, mid, W2, Wd, Ww1, Ww2, va, dist_emb, boundaries):
    src = edge_index[0]
    dst = edge_index[1]
    u = va[0] @ Ww1
    v = va[0] @ Ww2
    u1 = u[:D]
    u2 = u[D:]
    u3 = v[:D]
    u4 = v[D:]
    t = (dist_emb @ Wd.T) @ u2
    t16 = jnp.pad(t, (0, 1))
    b2 = jnp.broadcast_to((boundaries * boundaries)[:, None], (NB, 16))
    U3 = jnp.stack([u1, u3, u4], axis=1)
    degp = _deg_kernel(src).reshape(32, NPAD)[:, :N].T
    h2, scal = _dense_call(feat, W2.T, U3, degp)
    locf = loc.reshape(-1)
    scalf = scal.reshape(-1)
    parta = _edge_kernel_a(src, dst, mid, locf, h2, scalf, t16, b2)
    partb = _edge_kernel_b(src, dst, mid, locf, h2, scalf, t16, b2)
    return _fin_call(parta, partb, scal)


# speedup vs baseline: 8.0814x; 1.9518x over previous
import functools

import jax
import jax.numpy as jnp
from jax import lax
from jax.experimental import pallas as pl
from jax.experimental.pallas import tpu as pltpu
from jax.experimental.pallas import tpu_sc as plsc

N = 10000
NPAD = 10240
E = 320000
D = 128
NB = 14
QUART = 2500
ACC_ROWS = 2688
STRIPE = ACC_ROWS // 16
EPT = E // 16
K = 80
NCH = EPT // K
EPW = E // 32
NCHD = EPW // K

_mesh = plsc.VectorSubcoreMesh(core_axis_name="c", subcore_axis_name="s")
_sc_params = pltpu.CompilerParams(needs_layout_passes=False)


@functools.partial(
    pl.kernel,
    out_type=jax.ShapeDtypeStruct((32 * NPAD,), jnp.float32),
    mesh=_mesh,
    scratch_types=[
        pltpu.VMEM((K,), jnp.int32),
        pltpu.VMEM((NPAD,), jnp.float32),
    ],
    compiler_params=_sc_params,
)
def _deg_kernel(src_h, degp_h, idx_v, acc_v):
    cid = lax.axis_index("c")
    sid = lax.axis_index("s")
    wid = cid * 16 + sid
    zf = jnp.zeros((16,), jnp.float32)
    onef = jnp.ones((16,), jnp.float32)
    lane = lax.iota(jnp.int32, 16)

    @pl.loop(0, NPAD // 16)
    def _zero(i):
        acc_v[pl.ds(i * 16, 16)] = zf

    base0 = wid * EPW

    @pl.loop(0, NCHD)
    def _chunk(j):
        pltpu.sync_copy(src_h.at[pl.ds(base0 + j * K, K)], idx_v)
        for g in range(K // 16):
            iv = idx_v[pl.ds(g * 16, 16)]
            for l in range(16):
                plsc.addupdate_scatter(acc_v, [iv], onef, mask=lane == l)

    pltpu.sync_copy(acc_v, degp_h.at[pl.ds(wid * NPAD, NPAD)])


def _dense_body(feat_ref, w2t_ref, u3_ref, degt_ref, h2_ref, scal_ref):
    h2 = jnp.dot(feat_ref[...], w2t_ref[...], preferred_element_type=jnp.float32)
    h2_ref[...] = h2
    s = jnp.dot(h2, u3_ref[...], preferred_element_type=jnp.float32)
    deg = jnp.sum(degt_ref[...], axis=1, keepdims=True)
    d0 = lax.rsqrt(jnp.maximum(deg, 1.0))
    scal_ref[...] = jnp.concatenate([s, d0], axis=1)


_dense_call = pl.pallas_call(
    _dense_body,
    out_shape=(
        jax.ShapeDtypeStruct((N, D), jnp.float32),
        jax.ShapeDtypeStruct((N, 4), jnp.float32),
    ),
)


_edge_scratch = [
        pltpu.VMEM((4 * N,), jnp.float32),
        pltpu.VMEM((2 * N,), jnp.float32),
        pltpu.VMEM((16,), jnp.float32),
        pltpu.VMEM((NB, 16), jnp.float32),
        [pltpu.VMEM((K,), jnp.int32)] * 2,
        [pltpu.VMEM((K,), jnp.int32)] * 2,
        [pltpu.VMEM((K,), jnp.int32)] * 2,
        [pltpu.VMEM((K,), jnp.int32)] * 2,
        pltpu.VMEM((K,), jnp.float32),
        pltpu.VMEM((K,), jnp.float32),
        [pltpu.VMEM((K, D), jnp.float32)] * 2,
        [pltpu.VMEM((K, D), jnp.float32)] * 2,
        pltpu.VMEM_SHARED((ACC_ROWS, D), jnp.float32),
        [pltpu.SemaphoreType.DMA] * 2,
        [pltpu.SemaphoreType.DMA] * 2,
        [pltpu.SemaphoreType.DMA] * 2,
]


def _edge_body(lo, src_h, dst_h, mid_h, locf_h, h2_h, scalf_h, t_h, b2_h,
               part_h,
               scalf, locf, t_v, b2_v, srcb, dstb, midb, sdst, cabuf, cbbuf,
               asrc, amid, y_acc, semi, semg, sems):
    cid = lax.axis_index("c")
    sid = lax.axis_index("s")

    pltpu.sync_copy(scalf_h, scalf)
    pltpu.sync_copy(locf_h, locf)
    pltpu.sync_copy(t_h, t_v)
    pltpu.sync_copy(b2_h, b2_v)
    zf = jnp.zeros((16,), jnp.float32)

    @pl.loop(0, K * D // 16)
    def _zb(i):
        asrc[0][i // (D // 16), pl.ds((i % (D // 16)) * 16, 16)] = zf

    for r in range(STRIPE // K):
        pltpu.sync_copy(asrc[0], y_acc.at[pl.ds(sid * STRIPE + r * K, K)])
    pltpu.sync_copy(asrc[0].at[pl.ds(0, STRIPE % K)],
                    y_acc.at[pl.ds(sid * STRIPE + (STRIPE // K) * K, STRIPE % K)])
    plsc.subcore_barrier()

    base0 = sid * EPT
    lobase = lo + cid * QUART
    c0 = jnp.zeros((16,), jnp.int32)
    c1 = jnp.full((16,), 1, jnp.int32)
    c2 = jnp.full((16,), 2, jnp.int32)
    c3 = jnp.full((16,), 3, jnp.int32)
    chalf = jnp.full((16,), QUART, jnp.int32)
    cmask = jnp.full((16,), 127, jnp.int32)

    def _issue_idx(j, b):
        base = base0 + j * K
        pltpu.async_copy(src_h.at[pl.ds(base, K)], srcb[b], semi[b])
        pltpu.async_copy(dst_h.at[pl.ds(base, K)], dstb[b], semi[b])
        pltpu.async_copy(mid_h.at[pl.ds(base, K)], midb[b], semi[b])

    def _wait_idx(b):
        pltpu.make_async_copy(src_h.at[pl.ds(0, K)], srcb[b], semi[b]).wait()
        pltpu.make_async_copy(src_h.at[pl.ds(0, K)], dstb[b], semi[b]).wait()
        pltpu.make_async_copy(src_h.at[pl.ds(0, K)], midb[b], semi[b]).wait()

    def _issue_gather(b):
        pltpu.async_copy(h2_h.at[srcb[b]], asrc[b], semg[b])
        pltpu.async_copy(h2_h.at[midb[b]], amid[b], semg[b])

    def _wait_gather(b):
        pltpu.make_async_copy(h2_h.at[pl.ds(0, K)], asrc[b], semg[b]).wait()
        pltpu.make_async_copy(h2_h.at[pl.ds(0, K)], amid[b], semg[b]).wait()

    def _wait_scatter(b):
        pltpu.make_async_copy(h2_h.at[pl.ds(0, K)], asrc[b], sems[b]).wait()

    pltpu.sync_copy(src_h.at[pl.ds(base0, K)], srcb[0])
    pltpu.sync_copy(dst_h.at[pl.ds(base0, K)], dstb[0])
    pltpu.sync_copy(mid_h.at[pl.ds(base0, K)], midb[0])
    _issue_gather(0)

    @pl.loop(0, NCH // 2)
    def _pair(jj):
        for b in range(2):
            j = jj * 2 + b
            nb = 1 - b

            for g in range(K // 16):
                sv = srcb[b][pl.ds(g * 16, 16)]
                dv = dstb[b][pl.ds(g * 16, 16)]
                mv = midb[b][pl.ds(g * 16, 16)]
                sv2 = sv + sv
                dv2 = dv + dv
                lxs = plsc.load_gather(locf, [sv2])
                lys = plsc.load_gather(locf, [sv2 + c1])
                lxd = plsc.load_gather(locf, [dv2])
                lyd = plsc.load_gather(locf, [dv2 + c1])
                dx = lxd - lxs
                dy = lyd - lys
                d2 = dx * dx + dy * dy
                cnt = c0
                for q in range(NB):
                    cnt = cnt + (b2_v[q] < d2).astype(jnp.int32)
                tv = plsc.load_gather(t_v, [cnt])
                sv4 = sv2 + sv2
                dv4 = dv2 + dv2
                mv4 = (mv + mv) + (mv + mv)
                s1d = plsc.load_gather(scalf, [dv4])
                s3m = plsc.load_gather(scalf, [mv4 + c1])
                s4s = plsc.load_gather(scalf, [sv4 + c2])
                d0s = plsc.load_gather(scalf, [sv4 + c3])
                score = s1d + s4s + s3m + tv
                beta = 1.0 / (1.0 + jnp.exp(-score))
                cabuf[pl.ds(g * 16, 16)] = d0s * beta
                cbbuf[pl.ds(g * 16, 16)] = d0s
                rel = dv - lobase
                own = (rel >= c0) & (rel < chalf)
                sdst[b][pl.ds(g * 16, 16)] = jnp.where(
                    own, rel, chalf + (dv & cmask))

            @pl.when(j + 1 < NCH)
            def _pref_idx():
                _issue_idx(j + 1, nb)

            _wait_gather(b)

            @pl.loop(0, K)
            def _edge(k):
                kk = jnp.full((16,), k, jnp.int32)
                cav = plsc.load_gather(cabuf, [kk])
                cbv = plsc.load_gather(cbbuf, [kk])
                av = [asrc[b][k, pl.ds(s * 16, 16)] for s in range(D // 16)]
                bv = [amid[b][k, pl.ds(s * 16, 16)] for s in range(D // 16)]
                for s in range(D // 16):
                    asrc[b][k, pl.ds(s * 16, 16)] = cav * av[s] + cbv * bv[s]

            @pl.when(j + 1 < NCH)
            def _pref_gather():
                _wait_idx(nb)

                @pl.when(j >= 1)
                def _wsp():
                    _wait_scatter(nb)

                _issue_gather(nb)

            pltpu.async_copy(asrc[b], y_acc.at[sdst[b]], sems[b], add=True)

    _wait_scatter(0)
    _wait_scatter(1)
    plsc.subcore_barrier()
    pltpu.sync_copy(y_acc.at[pl.ds(sid * STRIPE, STRIPE)],
                    part_h.at[cid, pl.ds(sid * STRIPE, STRIPE)])


_edge_kernel_a = functools.partial(
    pl.kernel,
    out_type=jax.ShapeDtypeStruct((2, ACC_ROWS, D), jnp.float32),
    mesh=_mesh,
    scratch_types=_edge_scratch,
    compiler_params=_sc_params,
)(functools.partial(_edge_body, 0))

_edge_kernel_b = functools.partial(
    pl.kernel,
    out_type=jax.ShapeDtypeStruct((2, ACC_ROWS, D), jnp.float32),
    mesh=_mesh,
    scratch_types=_edge_scratch,
    compiler_params=_sc_params,
)(functools.partial(_edge_body, 2 * QUART))


def _fin_body(parta_ref, partb_ref, scal_ref, x_ref):
    y = jnp.concatenate(
        [parta_ref[0, :QUART, :], parta_ref[1, :QUART, :],
         partb_ref[0, :QUART, :], partb_ref[1, :QUART, :]], axis=0)
    x_ref[...] = scal_ref[:, 3:4] * y


_fin_call = pl.pallas_call(
    _fin_body,
    out_shape=jax.ShapeDtypeStruct((N, D), jnp.float32),
)


def kernel(feat, loc, edge_index, mid, W2, Wd, Ww1, Ww2, va, dist_emb, boundaries):
    src = edge_index[0]
    dst = edge_index[1]
    u = va[0] @ Ww1
    v = va[0] @ Ww2
    u1 = u[:D]
    u2 = u[D:]
    u3 = v[:D]
    u4 = v[D:]
    t = (dist_emb @ Wd.T) @ u2
    t16 = jnp.pad(t, (0, 1))
    b2 = jnp.broadcast_to((boundaries * boundaries)[:, None], (NB, 16))
    U3 = jnp.stack([u1, u3, u4], axis=1)
    degp = _deg_kernel(src).reshape(32, NPAD)[:, :N].T
    h2, scal = _dense_call(feat, W2.T, U3, degp)
    locf = loc.reshape(-1)
    scalf = scal.reshape(-1)
    parta = _edge_kernel_a(src, dst, mid, locf, h2, scalf, t16, b2)
    partb = _edge_kernel_b(src, dst, mid, locf, h2, scalf, t16, b2)
    return _fin_call(parta, partb, scal)


# speedup vs baseline: 20.4869x; 2.5351x over previous
import functools

import jax
import jax.numpy as jnp
from jax import lax
from jax.experimental import pallas as pl
from jax.experimental.pallas import tpu as pltpu
from jax.experimental.pallas import tpu_sc as plsc

N = 10000
NPAD = 10240
E = 320000
D = 128
NB = 14
HALFO = 5000
ACC_ROWS = 5120
STRIPE = ACC_ROWS // 16
EPT = E // 16
K = 80
NCH = EPT // K
EPW = E // 32
NCHD = EPW // K

_mesh = plsc.VectorSubcoreMesh(core_axis_name="c", subcore_axis_name="s")
_sc_params = pltpu.CompilerParams(needs_layout_passes=False)


@functools.partial(
    pl.kernel,
    out_type=jax.ShapeDtypeStruct((32 * NPAD,), jnp.float32),
    mesh=_mesh,
    scratch_types=[
        pltpu.VMEM((K,), jnp.int32),
        pltpu.VMEM((NPAD,), jnp.float32),
    ],
    compiler_params=_sc_params,
)
def _deg_kernel(src_h, degp_h, idx_v, acc_v):
    cid = lax.axis_index("c")
    sid = lax.axis_index("s")
    wid = cid * 16 + sid
    zf = jnp.zeros((16,), jnp.float32)
    onef = jnp.ones((16,), jnp.float32)
    lane = lax.iota(jnp.int32, 16)

    @pl.loop(0, NPAD // 16)
    def _zero(i):
        acc_v[pl.ds(i * 16, 16)] = zf

    base0 = wid * EPW

    @pl.loop(0, NCHD)
    def _chunk(j):
        pltpu.sync_copy(src_h.at[pl.ds(base0 + j * K, K)], idx_v)
        for g in range(K // 16):
            iv = idx_v[pl.ds(g * 16, 16)]
            for l in range(16):
                plsc.addupdate_scatter(acc_v, [iv], onef, mask=lane == l)

    pltpu.sync_copy(acc_v, degp_h.at[pl.ds(wid * NPAD, NPAD)])


def _dense_body(feat_ref, w2t_ref, u3_ref, degt_ref, h2_ref, scal_ref):
    h2 = jnp.dot(feat_ref[...], w2t_ref[...], preferred_element_type=jnp.float32)
    h2_ref[...] = h2
    s = jnp.dot(h2, u3_ref[...], preferred_element_type=jnp.float32)
    deg = jnp.sum(degt_ref[...], axis=1, keepdims=True)
    d0 = lax.rsqrt(jnp.maximum(deg, 1.0))
    scal_ref[...] = jnp.concatenate([s, d0], axis=1)


_dense_call = pl.pallas_call(
    _dense_body,
    out_shape=(
        jax.ShapeDtypeStruct((N, D), jnp.float32),
        jax.ShapeDtypeStruct((N, 4), jnp.float32),
    ),
)


_edge_scratch = [
        pltpu.VMEM((4 * N,), jnp.float32),
        pltpu.VMEM((2 * N,), jnp.float32),
        pltpu.VMEM((16,), jnp.float32),
        pltpu.VMEM((NB, 16), jnp.float32),
        [pltpu.VMEM((K,), jnp.int32)] * 2,
        [pltpu.VMEM((K,), jnp.int32)] * 2,
        [pltpu.VMEM((K,), jnp.int32)] * 2,
        [pltpu.VMEM((K,), jnp.int32)] * 2,
        pltpu.VMEM((K,), jnp.float32),
        pltpu.VMEM((K,), jnp.float32),
        pltpu.VMEM((K, D), jnp.float32),
        pltpu.VMEM((K, D), jnp.float32),
        pltpu.VMEM_SHARED((ACC_ROWS, D), jnp.float32),
        [pltpu.SemaphoreType.DMA] * 2,
        pltpu.SemaphoreType.DMA,
        pltpu.SemaphoreType.DMA,
]


def _edge_body(src_h, dst_h, mid_h, locf_h, h2_h, scalf_h, t_h, b2_h,
               part_h,
               scalf, locf, t_v, b2_v, srcb, dstb, midb, sdst, cabuf, cbbuf,
               asrc, amid, y_acc, semi, semg, sems):
    cid = lax.axis_index("c")
    sid = lax.axis_index("s")

    pltpu.sync_copy(scalf_h, scalf)
    pltpu.sync_copy(locf_h, locf)
    pltpu.sync_copy(t_h, t_v)
    pltpu.sync_copy(b2_h, b2_v)
    zf = jnp.zeros((16,), jnp.float32)

    @pl.loop(0, K * D // 16)
    def _zb(i):
        asrc[i // (D // 16), pl.ds((i % (D // 16)) * 16, 16)] = zf

    for r in range(STRIPE // K):
        pltpu.sync_copy(asrc, y_acc.at[pl.ds(sid * STRIPE + r * K, K)])
    pltpu.sync_copy(asrc.at[pl.ds(0, STRIPE % K)],
                    y_acc.at[pl.ds(sid * STRIPE + (STRIPE // K) * K, STRIPE % K)])
    plsc.subcore_barrier()

    base0 = sid * EPT
    lobase = cid * HALFO
    c0 = jnp.zeros((16,), jnp.int32)
    c1 = jnp.full((16,), 1, jnp.int32)
    c2 = jnp.full((16,), 2, jnp.int32)
    c3 = jnp.full((16,), 3, jnp.int32)
    chalf = jnp.full((16,), HALFO, jnp.int32)
    cmask = jnp.full((16,), 63, jnp.int32)

    def _issue_idx(j, b):
        base = base0 + j * K
        pltpu.async_copy(src_h.at[pl.ds(base, K)], srcb[b], semi[b])
        pltpu.async_copy(dst_h.at[pl.ds(base, K)], dstb[b], semi[b])
        pltpu.async_copy(mid_h.at[pl.ds(base, K)], midb[b], semi[b])

    def _wait_idx(b):
        pltpu.make_async_copy(src_h.at[pl.ds(0, K)], srcb[b], semi[b]).wait()
        pltpu.make_async_copy(src_h.at[pl.ds(0, K)], dstb[b], semi[b]).wait()
        pltpu.make_async_copy(src_h.at[pl.ds(0, K)], midb[b], semi[b]).wait()

    def _issue_gather(b):
        pltpu.async_copy(h2_h.at[srcb[b]], asrc, semg)
        pltpu.async_copy(h2_h.at[midb[b]], amid, semg)

    def _wait_gather():
        pltpu.make_async_copy(h2_h.at[pl.ds(0, K)], asrc, semg).wait()
        pltpu.make_async_copy(h2_h.at[pl.ds(0, K)], amid, semg).wait()

    def _wait_scatter():
        pltpu.make_async_copy(h2_h.at[pl.ds(0, K)], asrc, sems).wait()

    pltpu.sync_copy(src_h.at[pl.ds(base0, K)], srcb[0])
    pltpu.sync_copy(dst_h.at[pl.ds(base0, K)], dstb[0])
    pltpu.sync_copy(mid_h.at[pl.ds(base0, K)], midb[0])
    _issue_gather(0)

    @pl.loop(0, NCH // 2)
    def _pair(jj):
        for b in range(2):
            j = jj * 2 + b
            nb = 1 - b

            @pl.when(j >= 1)
            def _wis():
                _wait_idx(b)
                _wait_scatter()

            _issue_gather(b)

            for g in range(K // 16):
                sv = srcb[b][pl.ds(g * 16, 16)]
                dv = dstb[b][pl.ds(g * 16, 16)]
                mv = midb[b][pl.ds(g * 16, 16)]
                sv2 = sv + sv
                dv2 = dv + dv
                lxs = plsc.load_gather(locf, [sv2])
                lys = plsc.load_gather(locf, [sv2 + c1])
                lxd = plsc.load_gather(locf, [dv2])
                lyd = plsc.load_gather(locf, [dv2 + c1])
                dx = lxd - lxs
                dy = lyd - lys
                d2 = dx * dx + dy * dy
                cnt = c0
                for q in range(NB):
                    cnt = cnt + (b2_v[q] < d2).astype(jnp.int32)
                tv = plsc.load_gather(t_v, [cnt])
                sv4 = sv2 + sv2
                dv4 = dv2 + dv2
                mv4 = (mv + mv) + (mv + mv)
                s1d = plsc.load_gather(scalf, [dv4])
                s3m = plsc.load_gather(scalf, [mv4 + c1])
                s4s = plsc.load_gather(scalf, [sv4 + c2])
                d0s = plsc.load_gather(scalf, [sv4 + c3])
                score = s1d + s4s + s3m + tv
                beta = 1.0 / (1.0 + jnp.exp(-score))
                cabuf[pl.ds(g * 16, 16)] = d0s * beta
                cbbuf[pl.ds(g * 16, 16)] = d0s
                rel = dv - lobase
                own = (rel >= c0) & (rel < chalf)
                sdst[b][pl.ds(g * 16, 16)] = jnp.where(
                    own, rel, chalf + (dv & cmask))

            @pl.when(j + 1 < NCH)
            def _pref_idx():
                _issue_idx(j + 1, nb)

            _wait_gather()

            @pl.loop(0, K)
            def _edge(k):
                kk = jnp.full((16,), k, jnp.int32)
                cav = plsc.load_gather(cabuf, [kk])
                cbv = plsc.load_gather(cbbuf, [kk])
                av = [asrc[k, pl.ds(s * 16, 16)] for s in range(D // 16)]
                bv = [amid[k, pl.ds(s * 16, 16)] for s in range(D // 16)]
                for s in range(D // 16):
                    asrc[k, pl.ds(s * 16, 16)] = cav * av[s] + cbv * bv[s]

            pltpu.async_copy(asrc, y_acc.at[sdst[b]], sems, add=True)

    _wait_scatter()
    plsc.subcore_barrier()
    pltpu.sync_copy(y_acc.at[pl.ds(sid * STRIPE, STRIPE)],
                    part_h.at[cid, pl.ds(sid * STRIPE, STRIPE)])


_edge_kernel = functools.partial(
    pl.kernel,
    out_type=jax.ShapeDtypeStruct((2, ACC_ROWS, D), jnp.float32),
    mesh=_mesh,
    scratch_types=_edge_scratch,
    compiler_params=_sc_params,
)(_edge_body)


def _fin_body(part_ref, scal_ref, x_ref):
    y = jnp.concatenate(
        [part_ref[0, :HALFO, :], part_ref[1, :HALFO, :]], axis=0)
    x_ref[...] = scal_ref[:, 3:4] * y


_fin_call = pl.pallas_call(
    _fin_body,
    out_shape=jax.ShapeDtypeStruct((N, D), jnp.float32),
)


def kernel(feat, loc, edge_index, mid, W2, Wd, Ww1, Ww2, va, dist_emb, boundaries):
    src = edge_index[0]
    dst = edge_index[1]
    u = va[0] @ Ww1
    v = va[0] @ Ww2
    u1 = u[:D]
    u2 = u[D:]
    u3 = v[:D]
    u4 = v[D:]
    t = (dist_emb @ Wd.T) @ u2
    t16 = jnp.pad(t, (0, 1))
    b2 = jnp.broadcast_to((boundaries * boundaries)[:, None], (NB, 16))
    U3 = jnp.stack([u1, u3, u4], axis=1)
    degp = _deg_kernel(src).reshape(32, NPAD)[:, :N].T
    h2, scal = _dense_call(feat, W2.T, U3, degp)
    locf = loc.reshape(-1)
    scalf = scal.reshape(-1)
    part = _edge_kernel(src, dst, mid, locf, h2, scalf, t16, b2)
    return _fin_call(part, scal)
